# Initial kernel scaffold; baseline (speedup 1.0000x reference)
import jax
import jax.numpy as jnp
from jax.experimental import pallas as pl


def kernel(x, W1, b1, W2, b2, W3, b3):
    raise NotImplementedError("write your pallas kernel here")


# speedup vs baseline: 5.6446x; 5.6446x over previous
import functools

import jax
import jax.numpy as jnp
from jax import lax
from jax.experimental import pallas as pl
from jax.experimental.pallas import tpu as pltpu
from jax.experimental.pallas import tpu_sc as plsc

F32 = jnp.float32
I32 = jnp.int32


HIGHEST = jax.lax.Precision.HIGHEST


def _mm_kernel(h_ref, w_ref, o_ref):
    o_ref[...] = jnp.dot(h_ref[...], w_ref[...], preferred_element_type=F32)


def _matmul(h, W):
    m, d = h.shape
    bm = 1024
    return pl.pallas_call(
        _mm_kernel,
        grid=(m // bm,),
        in_specs=[
            pl.BlockSpec((bm, d), lambda i: (i, 0)),
            pl.BlockSpec((d, W.shape[1]), lambda i: (0, 0)),
        ],
        out_specs=pl.BlockSpec((bm, W.shape[1]), lambda i: (i, 0)),
        out_shape=jax.ShapeDtypeStruct((m, W.shape[1]), F32),
    )(h, W)


def _layer_kernel(hb_ref, hw_ref, b_ref, g_ref, idx_ref, *, n, dim, k, bi):
    i = pl.program_id(1)
    rowbase = i * bi

    hb = hb_ref[0]
    hb_i = hb_ref[0, pl.ds(rowbase, bi), :]
    hw = hw_ref[0]
    hw_i = hw_ref[0, pl.ds(rowbase, bi), :]

    sqj = jnp.sum(hb * hb, axis=1)
    sqi = jnp.sum(hb_i * hb_i, axis=1, keepdims=True)
    dot = lax.dot_general(hb_i, hb, (((1,), (1,)), ((), ())),
                          preferred_element_type=F32)
    d2 = sqi + sqj[None, :] - 2.0 * dot

    jidx = lax.broadcasted_iota(I32, (bi, n), 1)
    ridx = lax.broadcasted_iota(I32, (bi, n), 0) + rowbase
    big = jnp.array(1e30, F32)
    d2 = jnp.where(jidx == ridx, big, d2)

    macc = jnp.zeros((bi, n), F32)
    idx_rows = []
    for _ in range(k):
        m = jnp.min(d2, axis=1, keepdims=True)
        cand = jnp.where(d2 == m, jidx, n)
        jstar = jnp.min(cand, axis=1, keepdims=True)
        sel = jidx == jstar
        macc = jnp.where(sel, 1.0, macc)
        d2 = jnp.where(sel, big, d2)
        idx_rows.append(jstar.reshape(1, bi))
    idx_ref[0] = jnp.concatenate(idx_rows, axis=0)

    s = lax.dot_general(macc, hw, (((1,), (0,)), ((), ())),
                        preferred_element_type=F32,
                        precision=HIGHEST)
    g = (hw_i + s) * jnp.array(1.0 / 9.0, F32) + b_ref[0][None, :]
    g_ref[0] = jnp.maximum(g, 0.0)


def _tc_layer(hb, hw, b, *, nb, n, dim, k, bi):
    kern = functools.partial(_layer_kernel, n=n, dim=dim, k=k, bi=bi)
    return pl.pallas_call(
        kern,
        grid=(nb, n // bi),
        in_specs=[
            pl.BlockSpec((1, n, dim), lambda bq, i: (bq, 0, 0)),
            pl.BlockSpec((1, n, dim), lambda bq, i: (bq, 0, 0)),
            pl.BlockSpec((1, dim), lambda bq, i: (0, 0)),
        ],
        out_specs=[
            pl.BlockSpec((1, bi, dim), lambda bq, i: (bq, i, 0)),
            pl.BlockSpec((1, k, bi), lambda bq, i: (bq, 0, i)),
        ],
        out_shape=[
            jax.ShapeDtypeStruct((nb, n, dim), F32),
            jax.ShapeDtypeStruct((nb, k, n), I32),
        ],
    )(hb, hw, b.reshape(1, dim))


def _sc_body(idx_hbm, g_hbm, out_hbm, ibuf, gbuf, obuf, *, nb, n, dim, k,
             tiles_per_batch, rows_per_tile, ublk):
    cid = lax.axis_index("c")
    sid = lax.axis_index("s")
    wid = sid * 2 + cid
    bt = wid // tiles_per_batch
    rb = (wid % tiles_per_batch) * rows_per_tile

    nj = dim // 16

    def zero_row(r, _):
        for j in range(nj):
            obuf[r, pl.ds(j * 16, 16)] = jnp.zeros((16,), F32)
        return 0
    lax.fori_loop(0, rows_per_tile, zero_row, 0)

    nvec = ublk // 16

    def pbody(p, _):
        pbase = pl.multiple_of(p * ublk, ublk)
        pltpu.sync_copy(idx_hbm.at[bt, :, pl.ds(pbase, ublk)], ibuf)
        pltpu.sync_copy(g_hbm.at[pl.ds(bt * n + pbase, ublk), :], gbuf)

        def inner(it, _):
            kk = it // nvec
            c = it % nvec
            tvec = ibuf[kk, pl.ds(pl.multiple_of(c * 16, 16), 16)]
            for l in range(16):
                t = tvec[l]
                cond = jnp.logical_and(t >= rb, t < rb + rows_per_tile)

                @pl.when(cond)
                def _():
                    v = t - rb
                    u = c * 16 + l
                    for j in range(nj):
                        sl = pl.ds(j * 16, 16)
                        obuf[v, sl] = jnp.maximum(obuf[v, sl], gbuf[u, sl])
            return 0
        lax.fori_loop(0, k * nvec, inner, 0)
        return 0
    lax.fori_loop(0, n // ublk, pbody, 0)

    pltpu.sync_copy(obuf, out_hbm.at[pl.ds(bt * n + rb, rows_per_tile), :])


def _sc_scatter_max(idx, g, *, nb, n, dim, k, interpret=False):
    info_tiles = 32
    tiles_per_batch = info_tiles // nb
    rows_per_tile = n // tiles_per_batch
    ublk = 256
    mesh = plsc.VectorSubcoreMesh(core_axis_name="c", subcore_axis_name="s")
    body = functools.partial(
        _sc_body, nb=nb, n=n, dim=dim, k=k,
        tiles_per_batch=tiles_per_batch, rows_per_tile=rows_per_tile,
        ublk=ublk)
    return pl.kernel(
        body,
        out_type=jax.ShapeDtypeStruct((nb * n, dim), F32),
        mesh=mesh,
        scratch_types=[
            pltpu.VMEM((k, ublk), I32),
            pltpu.VMEM((ublk, dim), F32),
            pltpu.VMEM((rows_per_tile, dim), F32),
        ],
        interpret=interpret,
    )(idx, g)


def kernel(x, W1, b1, W2, b2, W3, b3):
    nb, c, n = x.shape
    dim = W1.shape[1]
    k = 8
    bi = 256
    nn = nb * n

    h = jnp.transpose(x, (0, 2, 1)).reshape(nn, c)
    h = jnp.pad(h, ((0, 0), (0, dim - c)))
    W1p = jnp.pad(W1, ((0, dim - c), (0, 0)))

    for W, b in ((W1p, b1), (W2, b2), (W3, b3)):
        hw = _matmul(h, W)
        g, idx = _tc_layer(h.reshape(nb, n, dim), hw.reshape(nb, n, dim), b,
                           nb=nb, n=n, dim=dim, k=k, bi=bi)
        h = _sc_scatter_max(idx, g.reshape(nn, dim), nb=nb, n=n, dim=dim, k=k)

    return jnp.transpose(h.reshape(nb, n, dim), (0, 2, 1))


# speedup vs baseline: 5.6553x; 1.0019x over previous
import functools

import jax
import jax.numpy as jnp
from jax import lax
from jax.experimental import pallas as pl
from jax.experimental.pallas import tpu as pltpu
from jax.experimental.pallas import tpu_sc as plsc

F32 = jnp.float32
I32 = jnp.int32


HIGHEST = jax.lax.Precision.HIGHEST


def _mm_kernel(h_ref, w_ref, o_ref):
    o_ref[...] = jnp.dot(h_ref[...], w_ref[...], preferred_element_type=F32)


def _matmul(h, W):
    m, d = h.shape
    bm = 1024
    return pl.pallas_call(
        _mm_kernel,
        grid=(m // bm,),
        in_specs=[
            pl.BlockSpec((bm, d), lambda i: (i, 0)),
            pl.BlockSpec((d, W.shape[1]), lambda i: (0, 0)),
        ],
        out_specs=pl.BlockSpec((bm, W.shape[1]), lambda i: (i, 0)),
        out_shape=jax.ShapeDtypeStruct((m, W.shape[1]), F32),
    )(h, W)


def _layer_kernel(hb_ref, hw_ref, b_ref, g_ref, idx_ref, *, n, dim, k, bi):
    i = pl.program_id(1)
    rowbase = i * bi

    hb = hb_ref[0]
    hb_i = hb_ref[0, pl.ds(rowbase, bi), :]
    hw = hw_ref[0]
    hw_i = hw_ref[0, pl.ds(rowbase, bi), :]

    sqj = jnp.sum(hb * hb, axis=1)
    sqi = jnp.sum(hb_i * hb_i, axis=1, keepdims=True)
    dot = lax.dot_general(hb_i, hb, (((1,), (1,)), ((), ())),
                          preferred_element_type=F32)
    d2 = sqi + sqj[None, :] - 2.0 * dot

    jidx = lax.broadcasted_iota(I32, (bi, n), 1)
    ridx = lax.broadcasted_iota(I32, (bi, n), 0) + rowbase
    big = jnp.array(1e30, F32)
    d2 = jnp.where(jidx == ridx, big, d2)

    macc = jnp.zeros((bi, n), F32)
    idx_rows = []
    for _ in range(k):
        m = jnp.min(d2, axis=1, keepdims=True)
        cand = jnp.where(d2 == m, jidx, n)
        jstar = jnp.min(cand, axis=1, keepdims=True)
        sel = jidx == jstar
        macc = jnp.where(sel, 1.0, macc)
        d2 = jnp.where(sel, big, d2)
        idx_rows.append(jstar.reshape(1, bi))
    idx_ref[0] = jnp.concatenate(idx_rows, axis=0)

    s = lax.dot_general(macc, hw, (((1,), (0,)), ((), ())),
                        preferred_element_type=F32,
                        precision=HIGHEST)
    g = (hw_i + s) * jnp.array(1.0 / 9.0, F32) + b_ref[0][None, :]
    g_ref[0] = jnp.maximum(g, 0.0)


def _tc_layer(hb, hw, b, *, nb, n, dim, k, bi):
    kern = functools.partial(_layer_kernel, n=n, dim=dim, k=k, bi=bi)
    return pl.pallas_call(
        kern,
        grid=(nb, n // bi),
        in_specs=[
            pl.BlockSpec((1, n, dim), lambda bq, i: (bq, 0, 0)),
            pl.BlockSpec((1, n, dim), lambda bq, i: (bq, 0, 0)),
            pl.BlockSpec((1, dim), lambda bq, i: (0, 0)),
        ],
        out_specs=[
            pl.BlockSpec((1, bi, dim), lambda bq, i: (bq, i, 0)),
            pl.BlockSpec((1, k, bi), lambda bq, i: (bq, 0, i)),
        ],
        out_shape=[
            jax.ShapeDtypeStruct((nb, n, dim), F32),
            jax.ShapeDtypeStruct((nb, k, n), I32),
        ],
    )(hb, hw, b.reshape(1, dim))


def _sc_body(idx_hbm, g_hbm, out_hbm, ibuf, gbuf, obuf, *,
             nb, n, dim, k, tiles_per_batch, rows_per_tile, ublk):
    cid = lax.axis_index("c")
    sid = lax.axis_index("s")
    wid = sid * 2 + cid
    bt = wid // tiles_per_batch
    rb = (wid % tiles_per_batch) * rows_per_tile

    nj = dim // 16
    sink = rows_per_tile

    def zero_row(r, _):
        for j in range(nj):
            obuf[r, pl.ds(j * 16, 16)] = jnp.zeros((16,), F32)
        return 0
    lax.fori_loop(0, rows_per_tile + 1, zero_row, 0)

    nvec = ublk // 16
    lane = lax.iota(I32, 16)

    def pbody(p, _):
        pbase = pl.multiple_of(p * ublk, ublk)
        pltpu.sync_copy(idx_hbm.at[bt, :, pl.ds(pbase, ublk)], ibuf)
        pltpu.sync_copy(g_hbm.at[pl.ds(bt * n + pbase, ublk), :], gbuf)

        def inner(it, _):
            kk = it // nvec
            c = it % nvec
            tvec = ibuf[kk, pl.ds(pl.multiple_of(c * 16, 16), 16)]
            for l in range(16):
                t = tvec[l]
                cond = jnp.logical_and(t >= rb, t < rb + rows_per_tile)
                v = jnp.where(cond, t - rb, sink)
                u = c * 16 + l
                for j in range(nj):
                    sl = pl.ds(j * 16, 16)
                    obuf[v, sl] = jnp.maximum(obuf[v, sl], gbuf[u, sl])
            return 0
        lax.fori_loop(0, k * nvec, inner, 0)
        return 0
    lax.fori_loop(0, n // ublk, pbody, 0)

    pltpu.sync_copy(obuf.at[pl.ds(0, rows_per_tile), :],
                    out_hbm.at[pl.ds(bt * n + rb, rows_per_tile), :])


def _sc_scatter_max(idx, g, *, nb, n, dim, k, interpret=False):
    info_tiles = 32
    tiles_per_batch = info_tiles // nb
    rows_per_tile = n // tiles_per_batch
    ublk = 256
    mesh = plsc.VectorSubcoreMesh(core_axis_name="c", subcore_axis_name="s")
    body = functools.partial(
        _sc_body, nb=nb, n=n, dim=dim, k=k,
        tiles_per_batch=tiles_per_batch, rows_per_tile=rows_per_tile,
        ublk=ublk)
    return pl.kernel(
        body,
        out_type=jax.ShapeDtypeStruct((nb * n, dim), F32),
        mesh=mesh,
        scratch_types=[
            pltpu.VMEM((k, ublk), I32),
            pltpu.VMEM((ublk, dim), F32),
            pltpu.VMEM((rows_per_tile + 1, dim), F32),
        ],
        interpret=interpret,
    )(idx, g)


def kernel(x, W1, b1, W2, b2, W3, b3):
    nb, c, n = x.shape
    dim = W1.shape[1]
    k = 8
    bi = 256
    nn = nb * n

    h = jnp.transpose(x, (0, 2, 1)).reshape(nn, c)
    h = jnp.pad(h, ((0, 0), (0, dim - c)))
    W1p = jnp.pad(W1, ((0, dim - c), (0, 0)))

    for W, b in ((W1p, b1), (W2, b2), (W3, b3)):
        hw = _matmul(h, W)
        g, idx = _tc_layer(h.reshape(nb, n, dim), hw.reshape(nb, n, dim), b,
                           nb=nb, n=n, dim=dim, k=k, bi=bi)
        h = _sc_scatter_max(idx, g.reshape(nn, dim), nb=nb, n=n, dim=dim, k=k)

    return jnp.transpose(h.reshape(nb, n, dim), (0, 2, 1))
